# baseline (speedup 1.0000x reference)
import functools

import jax
import jax.numpy as jnp
from jax import lax
from jax.experimental import pallas as pl
from jax.experimental.pallas import tpu as pltpu
from jax.experimental.pallas import tpu_sc as plsc

N_NODES = 10000
N_EDGES = 160000
D = 128
G = 4 * D
BLK = 2048
N_PAD = 10240
E_PAD = 163840


def _sigmoid(x):
  return 0.5 * jnp.tanh(0.5 * x) + 0.5


def _lstm_body(maxdeg_ref, deg_ref, hin_ref, wcat_ref, bias_ref, ws_ref,
               wn_ref, b_ref, msgs_ref, hout_ref, xh_ref, H_ref, C_ref,
               sem_ref, *, act):
  nb_nodes = N_PAD // BLK

  def zero_blk(i, _):
    z = jnp.zeros((BLK, D), jnp.float32)
    H_ref[pl.ds(i * BLK, BLK), :] = z
    C_ref[pl.ds(i * BLK, BLK), :] = z
    return 0
  lax.fori_loop(0, nb_nodes, zero_blk, 0)

  max_deg = maxdeg_ref[0]

  def issue(b, start):
    slot = lax.rem(b, 2)
    pltpu.make_async_copy(
        msgs_ref.at[pl.ds(start + b * BLK, BLK), :],
        xh_ref.at[slot, :, pl.ds(0, D)],
        sem_ref.at[slot],
    ).start()

  def wait(b, start):
    slot = lax.rem(b, 2)
    pltpu.make_async_copy(
        msgs_ref.at[pl.ds(start + b * BLK, BLK), :],
        xh_ref.at[slot, :, pl.ds(0, D)],
        sem_ref.at[slot],
    ).wait()

  def step(carry):
    t, start = carry
    deg = deg_ref[...]
    n_t = jnp.sum((deg > t).astype(jnp.int32))
    nb = (n_t + BLK - 1) // BLK
    issue(0, start)

    def blk(b, _):
      slot = lax.rem(b, 2)

      @pl.when(b + 1 < nb)
      def _():
        issue(b + 1, start)

      wait(b, start)
      r0 = b * BLK
      hp = H_ref[pl.ds(r0, BLK), :]
      cp = C_ref[pl.ds(r0, BLK), :]
      xh_ref[slot, :, pl.ds(D, D)] = hp
      xh = xh_ref[slot].astype(jnp.bfloat16)
      gates = jnp.dot(xh, wcat_ref[...],
                      preferred_element_type=jnp.float32) + bias_ref[...]
      i = _sigmoid(gates[:, 0:D])
      f = _sigmoid(gates[:, D:2 * D])
      g = jnp.tanh(gates[:, 2 * D:3 * D])
      o = _sigmoid(gates[:, 3 * D:4 * D])
      cn = f * cp + i * g
      hn = o * jnp.tanh(cn)
      mask = (r0 + lax.broadcasted_iota(jnp.int32, (BLK, 1), 0)) < n_t
      H_ref[pl.ds(r0, BLK), :] = jnp.where(mask, hn, hp)
      C_ref[pl.ds(r0, BLK), :] = jnp.where(mask, cn, cp)
      return 0

    lax.fori_loop(0, nb, blk, 0)
    return t + 1, start + n_t

  lax.while_loop(lambda c: c[0] < max_deg, step,
                 (jnp.int32(0), jnp.int32(0)))

  def out_blk(i, _):
    r0 = i * BLK
    hin = hin_ref[pl.ds(r0, BLK), :]
    hT = H_ref[pl.ds(r0, BLK), :]
    o = (jnp.dot(hin, ws_ref[...], preferred_element_type=jnp.float32)
         + jnp.dot(hT, wn_ref[...], preferred_element_type=jnp.float32)
         + b_ref[...])
    if act == "relu":
      o = jnp.maximum(o, 0.0)
    else:
      o = _sigmoid(o)
    hout_ref[pl.ds(r0, BLK), :] = o
    return 0
  lax.fori_loop(0, nb_nodes, out_blk, 0)


def _run_layer(msgs, hin, wcat, bias, ws, wn, b, deg2d, maxdeg, act):
  body = functools.partial(_lstm_body, act=act)
  return pl.pallas_call(
      body,
      out_shape=jax.ShapeDtypeStruct((N_PAD, D), jnp.float32),
      in_specs=[
          pl.BlockSpec(memory_space=pltpu.SMEM),
          pl.BlockSpec(memory_space=pltpu.VMEM),
          pl.BlockSpec(memory_space=pltpu.VMEM),
          pl.BlockSpec(memory_space=pltpu.VMEM),
          pl.BlockSpec(memory_space=pltpu.VMEM),
          pl.BlockSpec(memory_space=pltpu.VMEM),
          pl.BlockSpec(memory_space=pltpu.VMEM),
          pl.BlockSpec(memory_space=pltpu.VMEM),
          pl.BlockSpec(memory_space=pl.ANY),
      ],
      out_specs=pl.BlockSpec(memory_space=pltpu.VMEM),
      scratch_shapes=[
          pltpu.VMEM((2, BLK, 2 * D), jnp.float32),
          pltpu.VMEM((N_PAD, D), jnp.float32),
          pltpu.VMEM((N_PAD, D), jnp.float32),
          pltpu.SemaphoreType.DMA((2,)),
      ],
  )(maxdeg, deg2d, hin, wcat, bias, ws, wn, b, msgs)


def _preprocess(edge_index):
  src = edge_index[0]
  dst = edge_index[1]
  order = jnp.argsort(dst)
  src_s = src[order]
  dst_s = dst[order]
  deg = jnp.bincount(dst_s, length=N_NODES).astype(jnp.int32)
  offsets = jnp.concatenate(
      [jnp.zeros((1,), jnp.int32), jnp.cumsum(deg)[:-1].astype(jnp.int32)])
  perm = jnp.argsort(-deg)
  rank_of = jnp.zeros((N_NODES,), jnp.int32).at[perm].set(
      jnp.arange(N_NODES, dtype=jnp.int32))
  deg_sorted = deg[perm]

  cnt = jnp.bincount(deg, length=N_EDGES + 1)
  n_arr = (N_NODES - jnp.cumsum(cnt)).astype(jnp.int32)
  wsum = jnp.cumsum(jnp.arange(N_EDGES + 1) * cnt).astype(jnp.int32)

  tab_node = (offsets.astype(jnp.uint32) * 16384
              + rank_of.astype(jnp.uint32))
  by_dst = _sc_gather1(tab_node, dst_s)
  rank_dst = (by_dst % 16384).astype(jnp.int32)
  off_dst = (by_dst // 16384).astype(jnp.int32)
  rank_src = _sc_gather1(rank_of, src_s)
  t_e = jnp.arange(N_EDGES, dtype=jnp.int32) - off_dst
  tab_t = (n_arr.astype(jnp.uint32) * 262144
           + wsum.astype(jnp.uint32))
  by_t = _sc_gather1(tab_t, t_e)
  S_e = t_e * (by_t // 262144).astype(jnp.int32) + (
      by_t % 262144).astype(jnp.int32)
  p_e = S_e + rank_dst

  p_pad = jnp.concatenate(
      [p_e, jnp.arange(N_EDGES, E_PAD, dtype=jnp.int32)])
  v_pad = jnp.concatenate(
      [rank_src, jnp.zeros((E_PAD - N_EDGES,), jnp.int32)])
  g = _make_sc_scatter(E_PAD, 1280, E_PAD)(v_pad, p_pad)
  maxdeg = jnp.max(deg).reshape(1)
  deg2d = jnp.zeros((N_PAD,), jnp.int32).at[:N_NODES].set(
      deg_sorted).reshape(N_PAD // 128, 128)
  return g, perm, rank_of, deg2d, maxdeg


def _pad_params(Wi, Wh, bi, bh, Ws, Wn, b):
  din = Wi.shape[1]
  dout = Ws.shape[0]
  wcat = jnp.zeros((2 * D, G), jnp.float32)
  bias = jnp.zeros((G,), jnp.float32)
  for gi in range(4):
    wcat = wcat.at[0:din, gi * D:gi * D + din].set(
        Wi[gi * din:(gi + 1) * din, :].T)
    wcat = wcat.at[D:D + din, gi * D:gi * D + din].set(
        Wh[gi * din:(gi + 1) * din, :].T)
    bias = bias.at[gi * D:gi * D + din].set(
        bi[gi * din:(gi + 1) * din] + bh[gi * din:(gi + 1) * din])
  ws = jnp.zeros((D, D), jnp.float32).at[:din, :dout].set(Ws.T)
  wn = jnp.zeros((D, D), jnp.float32).at[:din, :dout].set(Wn.T)
  bp = jnp.zeros((D,), jnp.float32).at[:dout].set(b)
  return (wcat.astype(jnp.bfloat16), bias.reshape(1, G),
          ws, wn, bp.reshape(1, D))


_NW = 32


@functools.cache
def _make_sc_gather(B, CH, Dr, dtype):
  b_per_w = B // _NW
  nch = b_per_w // CH
  assert B % _NW == 0 and b_per_w % CH == 0 and CH % 8 == 0
  mesh = plsc.VectorSubcoreMesh(core_axis_name="c", subcore_axis_name="s")

  @functools.partial(
      pl.kernel, mesh=mesh,
      out_type=jax.ShapeDtypeStruct((B, Dr), dtype),
      scratch_types=[
          pltpu.VMEM((CH,), jnp.int32),
          pltpu.VMEM((CH,), jnp.int32),
          pltpu.VMEM((CH, Dr), dtype),
          pltpu.VMEM((CH, Dr), dtype),
          pltpu.SemaphoreType.DMA,
          pltpu.SemaphoreType.DMA,
      ],
  )
  def k(table_hbm, idx_hbm, out_hbm, idx0, idx1, rows0, rows1, sem0, sem1):
    wid = lax.axis_index("s") * 2 + lax.axis_index("c")
    base = wid * b_per_w
    idx_v = [idx0, idx1]
    rows_v = [rows0, rows1]
    sem = [sem0, sem1]

    def start_gather(c):
      s = c % 2
      pltpu.sync_copy(idx_hbm.at[pl.ds(base + c * CH, CH)], idx_v[s])
      pltpu.make_async_copy(table_hbm.at[idx_v[s]], rows_v[s],
                            sem[s]).start()

    start_gather(0)
    for c in range(nch):
      s = c % 2
      if c + 1 < nch:
        start_gather(c + 1)
      pltpu.make_async_copy(table_hbm.at[idx_v[s]], rows_v[s],
                            sem[s]).wait()
      pltpu.sync_copy(rows_v[s], out_hbm.at[pl.ds(base + c * CH, CH)])

  return k


@functools.cache
def _make_sc_scatter(B, CH, M):
  b_per_w = B // _NW
  nch = b_per_w // CH
  assert B % _NW == 0 and b_per_w % CH == 0 and CH % 8 == 0
  mesh = plsc.VectorSubcoreMesh(core_axis_name="c", subcore_axis_name="s")

  @functools.partial(
      pl.kernel, mesh=mesh,
      out_type=jax.ShapeDtypeStruct((M,), jnp.int32),
      scratch_types=[
          pltpu.VMEM((CH,), jnp.int32),
          pltpu.VMEM((CH,), jnp.int32),
          pltpu.VMEM((CH,), jnp.int32),
          pltpu.VMEM((CH,), jnp.int32),
          pltpu.SemaphoreType.DMA,
          pltpu.SemaphoreType.DMA,
      ],
  )
  def k(vals_hbm, pos_hbm, out_hbm, pos0, pos1, rows0, rows1, sem0, sem1):
    wid = lax.axis_index("s") * 2 + lax.axis_index("c")
    base = wid * b_per_w
    pos_v = [pos0, pos1]
    rows_v = [rows0, rows1]
    sem = [sem0, sem1]

    def start_scatter(c):
      s = c % 2
      pltpu.sync_copy(pos_hbm.at[pl.ds(base + c * CH, CH)], pos_v[s])
      pltpu.sync_copy(vals_hbm.at[pl.ds(base + c * CH, CH)], rows_v[s])
      pltpu.make_async_copy(rows_v[s], out_hbm.at[pos_v[s]], sem[s]).start()

    start_scatter(0)
    for c in range(nch):
      s = c % 2
      if c + 1 < nch:
        start_scatter(c + 1)
      pltpu.make_async_copy(rows_v[s], out_hbm.at[pos_v[s]], sem[s]).wait()

  return k


def _gather_rows(table, idx):
  return _make_sc_gather(idx.shape[0],
                         320 if idx.shape[0] == N_PAD else 320, D,
                         jnp.float32)(table, idx)


@functools.cache
def _make_sc_gather1(B, CH, dtype):
  b_per_w = B // _NW
  nch = b_per_w // CH
  assert B % _NW == 0 and b_per_w % CH == 0 and CH % 8 == 0
  mesh = plsc.VectorSubcoreMesh(core_axis_name="c", subcore_axis_name="s")

  @functools.partial(
      pl.kernel, mesh=mesh,
      out_type=jax.ShapeDtypeStruct((B,), dtype),
      scratch_types=[
          pltpu.VMEM((CH,), jnp.int32),
          pltpu.VMEM((CH,), jnp.int32),
          pltpu.VMEM((CH,), dtype),
          pltpu.VMEM((CH,), dtype),
          pltpu.SemaphoreType.DMA,
          pltpu.SemaphoreType.DMA,
      ],
  )
  def k(table_hbm, idx_hbm, out_hbm, idx0, idx1, rows0, rows1, sem0, sem1):
    wid = lax.axis_index("s") * 2 + lax.axis_index("c")
    base = wid * b_per_w
    idx_v = [idx0, idx1]
    rows_v = [rows0, rows1]
    sem = [sem0, sem1]

    def start_gather(c):
      s = c % 2
      pltpu.sync_copy(idx_hbm.at[pl.ds(base + c * CH, CH)], idx_v[s])
      pltpu.make_async_copy(table_hbm.at[idx_v[s]], rows_v[s],
                            sem[s]).start()

    start_gather(0)
    for c in range(nch):
      s = c % 2
      if c + 1 < nch:
        start_gather(c + 1)
      pltpu.make_async_copy(table_hbm.at[idx_v[s]], rows_v[s],
                            sem[s]).wait()
      pltpu.sync_copy(rows_v[s], out_hbm.at[pl.ds(base + c * CH, CH)])

  return k


def _sc_gather1(table, idx, ch=1000):
  return _make_sc_gather1(idx.shape[0], ch, table.dtype)(table, idx)


def kernel(x, edge_index, l1_Wi, l1_Wh, l1_bi, l1_bh, l1_Ws, l1_Wn, l1_b,
           l2_Wi, l2_Wh, l2_bi, l2_bh, l2_Ws, l2_Wn, l2_b,
           l3_Wi, l3_Wh, l3_bi, l3_bh, l3_Ws, l3_Wn, l3_b,
           l4_Wi, l4_Wh, l4_bi, l4_bh, l4_Ws, l4_Wn, l4_b):
  g, perm, rank_of, deg2d, maxdeg = _preprocess(edge_index)

  params = [
      _pad_params(l1_Wi, l1_Wh, l1_bi, l1_bh, l1_Ws, l1_Wn, l1_b),
      _pad_params(l2_Wi, l2_Wh, l2_bi, l2_bh, l2_Ws, l2_Wn, l2_b),
      _pad_params(l3_Wi, l3_Wh, l3_bi, l3_bh, l3_Ws, l3_Wn, l3_b),
      _pad_params(l4_Wi, l4_Wh, l4_bi, l4_bh, l4_Ws, l4_Wn, l4_b),
  ]
  acts = ["relu", "relu", "relu", "sigmoid"]

  perm_pad = jnp.zeros((N_PAD,), jnp.int32).at[:N_NODES].set(perm)
  h = jnp.zeros((N_PAD, D), jnp.float32).at[:N_NODES, :x.shape[1]].set(x)
  h = _gather_rows(h, perm_pad)

  for li in range(4):
    wcat, bias, ws, wn, bp = params[li]
    msgs = _gather_rows(h, g)
    h = _run_layer(msgs, h, wcat, bias, ws, wn, bp, deg2d, maxdeg, acts[li])

  rank_pad = jnp.zeros((N_PAD,), jnp.int32).at[:N_NODES].set(rank_of)
  out = _gather_rows(h, rank_pad)
  return out[:N_NODES, :1]


# speedup vs baseline: 1.0126x; 1.0126x over previous
import functools

import jax
import jax.numpy as jnp
from jax import lax
from jax.experimental import pallas as pl
from jax.experimental.pallas import tpu as pltpu
from jax.experimental.pallas import tpu_sc as plsc

N_NODES = 10000
N_EDGES = 160000
D = 128
G = 4 * D
BLK = 2048
N_PAD = 10240
E_PAD = 163840


def _sigmoid(x):
  return 0.5 * jnp.tanh(0.5 * x) + 0.5


def _lstm_body(maxdeg_ref, deg_ref, hin_ref, wcat_ref, bias_ref, ws_ref,
               wn_ref, b_ref, msgs_ref, hout_ref, xh_ref, H_ref, C_ref,
               sem_ref, *, act):
  nb_nodes = N_PAD // BLK

  def zero_blk(i, _):
    z = jnp.zeros((BLK, D), jnp.float32)
    H_ref[pl.ds(i * BLK, BLK), :] = z
    C_ref[pl.ds(i * BLK, BLK), :] = z
    return 0
  lax.fori_loop(0, nb_nodes, zero_blk, 0)

  max_deg = maxdeg_ref[0]

  def issue(b, start):
    slot = lax.rem(b, 2)
    pltpu.make_async_copy(
        msgs_ref.at[pl.ds(start + b * BLK, BLK), :],
        xh_ref.at[slot, :, pl.ds(0, D)],
        sem_ref.at[slot],
    ).start()

  def wait(b, start):
    slot = lax.rem(b, 2)
    pltpu.make_async_copy(
        msgs_ref.at[pl.ds(start + b * BLK, BLK), :],
        xh_ref.at[slot, :, pl.ds(0, D)],
        sem_ref.at[slot],
    ).wait()

  def step(carry):
    t, start = carry
    deg = deg_ref[...]
    n_t = jnp.sum((deg > t).astype(jnp.int32))
    nb = (n_t + BLK - 1) // BLK
    issue(0, start)

    def blk(b, _):
      slot = lax.rem(b, 2)

      @pl.when(b + 1 < nb)
      def _():
        issue(b + 1, start)

      wait(b, start)
      r0 = b * BLK
      hp = H_ref[pl.ds(r0, BLK), :]
      cp = C_ref[pl.ds(r0, BLK), :]
      xh_ref[slot, :, pl.ds(D, D)] = hp
      xh = xh_ref[slot].astype(jnp.bfloat16)
      gates = jnp.dot(xh, wcat_ref[...],
                      preferred_element_type=jnp.float32) + bias_ref[...]
      i = _sigmoid(gates[:, 0:D])
      f = _sigmoid(gates[:, D:2 * D])
      g = jnp.tanh(gates[:, 2 * D:3 * D])
      o = _sigmoid(gates[:, 3 * D:4 * D])
      cn = f * cp + i * g
      hn = o * jnp.tanh(cn)
      mask = (r0 + lax.broadcasted_iota(jnp.int32, (BLK, 1), 0)) < n_t
      H_ref[pl.ds(r0, BLK), :] = jnp.where(mask, hn, hp)
      C_ref[pl.ds(r0, BLK), :] = jnp.where(mask, cn, cp)
      return 0

    lax.fori_loop(0, nb, blk, 0)
    return t + 1, start + n_t

  lax.while_loop(lambda c: c[0] < max_deg, step,
                 (jnp.int32(0), jnp.int32(0)))

  def out_blk(i, _):
    r0 = i * BLK
    hin = hin_ref[pl.ds(r0, BLK), :]
    hT = H_ref[pl.ds(r0, BLK), :]
    o = (jnp.dot(hin, ws_ref[...], preferred_element_type=jnp.float32)
         + jnp.dot(hT, wn_ref[...], preferred_element_type=jnp.float32)
         + b_ref[...])
    if act == "relu":
      o = jnp.maximum(o, 0.0)
    else:
      o = _sigmoid(o)
    hout_ref[pl.ds(r0, BLK), :] = o
    return 0
  lax.fori_loop(0, nb_nodes, out_blk, 0)


def _run_layer(msgs, hin, wcat, bias, ws, wn, b, deg2d, maxdeg, act):
  body = functools.partial(_lstm_body, act=act)
  return pl.pallas_call(
      body,
      out_shape=jax.ShapeDtypeStruct((N_PAD, D), jnp.float32),
      in_specs=[
          pl.BlockSpec(memory_space=pltpu.SMEM),
          pl.BlockSpec(memory_space=pltpu.VMEM),
          pl.BlockSpec(memory_space=pltpu.VMEM),
          pl.BlockSpec(memory_space=pltpu.VMEM),
          pl.BlockSpec(memory_space=pltpu.VMEM),
          pl.BlockSpec(memory_space=pltpu.VMEM),
          pl.BlockSpec(memory_space=pltpu.VMEM),
          pl.BlockSpec(memory_space=pltpu.VMEM),
          pl.BlockSpec(memory_space=pl.ANY),
      ],
      out_specs=pl.BlockSpec(memory_space=pltpu.VMEM),
      scratch_shapes=[
          pltpu.VMEM((2, BLK, 2 * D), jnp.float32),
          pltpu.VMEM((N_PAD, D), jnp.float32),
          pltpu.VMEM((N_PAD, D), jnp.float32),
          pltpu.SemaphoreType.DMA((2,)),
      ],
  )(maxdeg, deg2d, hin, wcat, bias, ws, wn, b, msgs)


def _preprocess(edge_index):
  src = edge_index[0]
  dst = edge_index[1]
  order = jnp.argsort(dst)
  src_s = src[order]
  dst_s = dst[order]
  deg = jnp.bincount(dst_s, length=N_NODES).astype(jnp.int32)
  offsets = jnp.concatenate(
      [jnp.zeros((1,), jnp.int32), jnp.cumsum(deg)[:-1].astype(jnp.int32)])
  perm = jnp.argsort(-deg)
  rank_of = jnp.zeros((N_NODES,), jnp.int32).at[perm].set(
      jnp.arange(N_NODES, dtype=jnp.int32))
  deg_sorted = deg[perm]

  cnt = jnp.bincount(deg, length=N_EDGES + 1)
  n_arr = (N_NODES - jnp.cumsum(cnt)).astype(jnp.int32)
  wsum = jnp.cumsum(jnp.arange(N_EDGES + 1) * cnt).astype(jnp.int32)

  tab_node = (offsets.astype(jnp.uint32) * 16384
              + rank_of.astype(jnp.uint32))
  starts = jnp.where(deg > 0, offsets, N_EDGES)
  seg = jnp.zeros((N_EDGES,), jnp.uint32).at[starts].set(tab_node)
  by_dst = lax.cummax(seg)
  rank_dst = (by_dst % 16384).astype(jnp.int32)
  off_dst = (by_dst // 16384).astype(jnp.int32)
  rank_src = _sc_gather1(rank_of, src_s)
  t_e = jnp.arange(N_EDGES, dtype=jnp.int32) - off_dst
  tab_t = (n_arr.astype(jnp.uint32) * 262144
           + wsum.astype(jnp.uint32))
  by_t = _sc_gather1(tab_t, t_e)
  S_e = t_e * (by_t // 262144).astype(jnp.int32) + (
      by_t % 262144).astype(jnp.int32)
  p_e = S_e + rank_dst

  p_pad = jnp.concatenate(
      [p_e, jnp.arange(N_EDGES, E_PAD, dtype=jnp.int32)])
  v_pad = jnp.concatenate(
      [rank_src, jnp.zeros((E_PAD - N_EDGES,), jnp.int32)])
  g = _make_sc_scatter(E_PAD, 1280, E_PAD)(v_pad, p_pad)
  maxdeg = jnp.max(deg).reshape(1)
  deg2d = jnp.zeros((N_PAD,), jnp.int32).at[:N_NODES].set(
      deg_sorted).reshape(N_PAD // 128, 128)
  return g, perm, rank_of, deg2d, maxdeg


def _pad_params(Wi, Wh, bi, bh, Ws, Wn, b):
  din = Wi.shape[1]
  dout = Ws.shape[0]
  wcat = jnp.zeros((2 * D, G), jnp.float32)
  bias = jnp.zeros((G,), jnp.float32)
  for gi in range(4):
    wcat = wcat.at[0:din, gi * D:gi * D + din].set(
        Wi[gi * din:(gi + 1) * din, :].T)
    wcat = wcat.at[D:D + din, gi * D:gi * D + din].set(
        Wh[gi * din:(gi + 1) * din, :].T)
    bias = bias.at[gi * D:gi * D + din].set(
        bi[gi * din:(gi + 1) * din] + bh[gi * din:(gi + 1) * din])
  ws = jnp.zeros((D, D), jnp.float32).at[:din, :dout].set(Ws.T)
  wn = jnp.zeros((D, D), jnp.float32).at[:din, :dout].set(Wn.T)
  bp = jnp.zeros((D,), jnp.float32).at[:dout].set(b)
  return (wcat.astype(jnp.bfloat16), bias.reshape(1, G),
          ws, wn, bp.reshape(1, D))


_NW = 32


@functools.cache
def _make_sc_gather(B, CH, Dr, dtype):
  b_per_w = B // _NW
  nch = b_per_w // CH
  assert B % _NW == 0 and b_per_w % CH == 0 and CH % 8 == 0
  mesh = plsc.VectorSubcoreMesh(core_axis_name="c", subcore_axis_name="s")

  @functools.partial(
      pl.kernel, mesh=mesh,
      out_type=jax.ShapeDtypeStruct((B, Dr), dtype),
      scratch_types=[
          pltpu.VMEM((CH,), jnp.int32),
          pltpu.VMEM((CH,), jnp.int32),
          pltpu.VMEM((CH, Dr), dtype),
          pltpu.VMEM((CH, Dr), dtype),
          pltpu.SemaphoreType.DMA,
          pltpu.SemaphoreType.DMA,
      ],
  )
  def k(table_hbm, idx_hbm, out_hbm, idx0, idx1, rows0, rows1, sem0, sem1):
    wid = lax.axis_index("s") * 2 + lax.axis_index("c")
    base = wid * b_per_w
    idx_v = [idx0, idx1]
    rows_v = [rows0, rows1]
    sem = [sem0, sem1]

    def start_gather(c):
      s = c % 2
      pltpu.sync_copy(idx_hbm.at[pl.ds(base + c * CH, CH)], idx_v[s])
      pltpu.make_async_copy(table_hbm.at[idx_v[s]], rows_v[s],
                            sem[s]).start()

    start_gather(0)
    for c in range(nch):
      s = c % 2
      if c + 1 < nch:
        start_gather(c + 1)
      pltpu.make_async_copy(table_hbm.at[idx_v[s]], rows_v[s],
                            sem[s]).wait()
      pltpu.sync_copy(rows_v[s], out_hbm.at[pl.ds(base + c * CH, CH)])

  return k


@functools.cache
def _make_sc_scatter(B, CH, M):
  b_per_w = B // _NW
  nch = b_per_w // CH
  assert B % _NW == 0 and b_per_w % CH == 0 and CH % 8 == 0
  mesh = plsc.VectorSubcoreMesh(core_axis_name="c", subcore_axis_name="s")

  @functools.partial(
      pl.kernel, mesh=mesh,
      out_type=jax.ShapeDtypeStruct((M,), jnp.int32),
      scratch_types=[
          pltpu.VMEM((CH,), jnp.int32),
          pltpu.VMEM((CH,), jnp.int32),
          pltpu.VMEM((CH,), jnp.int32),
          pltpu.VMEM((CH,), jnp.int32),
          pltpu.SemaphoreType.DMA,
          pltpu.SemaphoreType.DMA,
      ],
  )
  def k(vals_hbm, pos_hbm, out_hbm, pos0, pos1, rows0, rows1, sem0, sem1):
    wid = lax.axis_index("s") * 2 + lax.axis_index("c")
    base = wid * b_per_w
    pos_v = [pos0, pos1]
    rows_v = [rows0, rows1]
    sem = [sem0, sem1]

    def start_scatter(c):
      s = c % 2
      pltpu.sync_copy(pos_hbm.at[pl.ds(base + c * CH, CH)], pos_v[s])
      pltpu.sync_copy(vals_hbm.at[pl.ds(base + c * CH, CH)], rows_v[s])
      pltpu.make_async_copy(rows_v[s], out_hbm.at[pos_v[s]], sem[s]).start()

    start_scatter(0)
    for c in range(nch):
      s = c % 2
      if c + 1 < nch:
        start_scatter(c + 1)
      pltpu.make_async_copy(rows_v[s], out_hbm.at[pos_v[s]], sem[s]).wait()

  return k


def _gather_rows(table, idx):
  return _make_sc_gather(idx.shape[0],
                         320 if idx.shape[0] == N_PAD else 320, D,
                         jnp.float32)(table, idx)


@functools.cache
def _make_sc_gather1(B, CH, dtype):
  b_per_w = B // _NW
  nch = b_per_w // CH
  assert B % _NW == 0 and b_per_w % CH == 0 and CH % 8 == 0
  mesh = plsc.VectorSubcoreMesh(core_axis_name="c", subcore_axis_name="s")

  @functools.partial(
      pl.kernel, mesh=mesh,
      out_type=jax.ShapeDtypeStruct((B,), dtype),
      scratch_types=[
          pltpu.VMEM((CH,), jnp.int32),
          pltpu.VMEM((CH,), jnp.int32),
          pltpu.VMEM((CH,), dtype),
          pltpu.VMEM((CH,), dtype),
          pltpu.SemaphoreType.DMA,
          pltpu.SemaphoreType.DMA,
      ],
  )
  def k(table_hbm, idx_hbm, out_hbm, idx0, idx1, rows0, rows1, sem0, sem1):
    wid = lax.axis_index("s") * 2 + lax.axis_index("c")
    base = wid * b_per_w
    idx_v = [idx0, idx1]
    rows_v = [rows0, rows1]
    sem = [sem0, sem1]

    def start_gather(c):
      s = c % 2
      pltpu.sync_copy(idx_hbm.at[pl.ds(base + c * CH, CH)], idx_v[s])
      pltpu.make_async_copy(table_hbm.at[idx_v[s]], rows_v[s],
                            sem[s]).start()

    start_gather(0)
    for c in range(nch):
      s = c % 2
      if c + 1 < nch:
        start_gather(c + 1)
      pltpu.make_async_copy(table_hbm.at[idx_v[s]], rows_v[s],
                            sem[s]).wait()
      pltpu.sync_copy(rows_v[s], out_hbm.at[pl.ds(base + c * CH, CH)])

  return k


def _sc_gather1(table, idx, ch=1000):
  return _make_sc_gather1(idx.shape[0], ch, table.dtype)(table, idx)


def kernel(x, edge_index, l1_Wi, l1_Wh, l1_bi, l1_bh, l1_Ws, l1_Wn, l1_b,
           l2_Wi, l2_Wh, l2_bi, l2_bh, l2_Ws, l2_Wn, l2_b,
           l3_Wi, l3_Wh, l3_bi, l3_bh, l3_Ws, l3_Wn, l3_b,
           l4_Wi, l4_Wh, l4_bi, l4_bh, l4_Ws, l4_Wn, l4_b):
  g, perm, rank_of, deg2d, maxdeg = _preprocess(edge_index)

  params = [
      _pad_params(l1_Wi, l1_Wh, l1_bi, l1_bh, l1_Ws, l1_Wn, l1_b),
      _pad_params(l2_Wi, l2_Wh, l2_bi, l2_bh, l2_Ws, l2_Wn, l2_b),
      _pad_params(l3_Wi, l3_Wh, l3_bi, l3_bh, l3_Ws, l3_Wn, l3_b),
      _pad_params(l4_Wi, l4_Wh, l4_bi, l4_bh, l4_Ws, l4_Wn, l4_b),
  ]
  acts = ["relu", "relu", "relu", "sigmoid"]

  perm_pad = jnp.zeros((N_PAD,), jnp.int32).at[:N_NODES].set(perm)
  h = jnp.zeros((N_PAD, D), jnp.float32).at[:N_NODES, :x.shape[1]].set(x)
  h = _gather_rows(h, perm_pad)

  for li in range(4):
    wcat, bias, ws, wn, bp = params[li]
    msgs = _gather_rows(h, g)
    h = _run_layer(msgs, h, wcat, bias, ws, wn, bp, deg2d, maxdeg, acts[li])

  rank_pad = jnp.zeros((N_PAD,), jnp.int32).at[:N_NODES].set(rank_of)
  out = _gather_rows(h, rank_pad)
  return out[:N_NODES, :1]


# speedup vs baseline: 1.0151x; 1.0025x over previous
import functools

import jax
import jax.numpy as jnp
from jax import lax
from jax.experimental import pallas as pl
from jax.experimental.pallas import tpu as pltpu
from jax.experimental.pallas import tpu_sc as plsc

N_NODES = 10000
N_EDGES = 160000
D = 128
G = 4 * D
BLK = 2048
N_PAD = 10240
E_PAD = 163840


def _sigmoid(x):
  return 0.5 * jnp.tanh(0.5 * x) + 0.5


def _lstm_body(maxdeg_ref, deg_ref, hin_ref, wcat_ref, bias_ref, ws_ref,
               wn_ref, b_ref, msgs_ref, hout_ref, xh_ref, H_ref, C_ref,
               sem_ref, *, act):
  nb_nodes = N_PAD // BLK

  def zero_blk(i, _):
    z = jnp.zeros((BLK, D), jnp.float32)
    H_ref[pl.ds(i * BLK, BLK), :] = z
    C_ref[pl.ds(i * BLK, BLK), :] = z
    return 0
  lax.fori_loop(0, nb_nodes, zero_blk, 0)

  max_deg = maxdeg_ref[0]

  def issue(b, start):
    slot = lax.rem(b, 2)
    pltpu.make_async_copy(
        msgs_ref.at[pl.ds(start + b * BLK, BLK), :],
        xh_ref.at[slot, :, pl.ds(0, D)],
        sem_ref.at[slot],
    ).start()

  def wait(b, start):
    slot = lax.rem(b, 2)
    pltpu.make_async_copy(
        msgs_ref.at[pl.ds(start + b * BLK, BLK), :],
        xh_ref.at[slot, :, pl.ds(0, D)],
        sem_ref.at[slot],
    ).wait()

  def step(carry):
    t, start = carry
    deg = deg_ref[...]
    n_t = jnp.sum((deg > t).astype(jnp.int32))
    nb = (n_t + BLK - 1) // BLK
    issue(0, start)

    def blk(b, _):
      slot = lax.rem(b, 2)

      @pl.when(b + 1 < nb)
      def _():
        issue(b + 1, start)

      wait(b, start)
      r0 = b * BLK
      hp = H_ref[pl.ds(r0, BLK), :]
      cp = C_ref[pl.ds(r0, BLK), :]
      xh_ref[slot, :, pl.ds(D, D)] = hp
      xh = xh_ref[slot].astype(jnp.bfloat16)
      gates = jnp.dot(xh, wcat_ref[...],
                      preferred_element_type=jnp.float32) + bias_ref[...]
      i = _sigmoid(gates[:, 0:D])
      f = _sigmoid(gates[:, D:2 * D])
      g = jnp.tanh(gates[:, 2 * D:3 * D])
      o = _sigmoid(gates[:, 3 * D:4 * D])
      cn = f * cp + i * g
      hn = o * jnp.tanh(cn)
      mask = (r0 + lax.broadcasted_iota(jnp.int32, (BLK, 1), 0)) < n_t
      H_ref[pl.ds(r0, BLK), :] = jnp.where(mask, hn, hp)
      C_ref[pl.ds(r0, BLK), :] = jnp.where(mask, cn, cp)
      return 0

    lax.fori_loop(0, nb, blk, 0)
    return t + 1, start + n_t

  lax.while_loop(lambda c: c[0] < max_deg, step,
                 (jnp.int32(0), jnp.int32(0)))

  def out_blk(i, _):
    r0 = i * BLK
    hin = hin_ref[pl.ds(r0, BLK), :]
    hT = H_ref[pl.ds(r0, BLK), :]
    o = (jnp.dot(hin, ws_ref[...], preferred_element_type=jnp.float32)
         + jnp.dot(hT, wn_ref[...], preferred_element_type=jnp.float32)
         + b_ref[...])
    if act == "relu":
      o = jnp.maximum(o, 0.0)
    else:
      o = _sigmoid(o)
    hout_ref[pl.ds(r0, BLK), :] = o
    return 0
  lax.fori_loop(0, nb_nodes, out_blk, 0)


def _run_layer(msgs, hin, wcat, bias, ws, wn, b, deg2d, maxdeg, act):
  body = functools.partial(_lstm_body, act=act)
  return pl.pallas_call(
      body,
      out_shape=jax.ShapeDtypeStruct((N_PAD, D), jnp.float32),
      in_specs=[
          pl.BlockSpec(memory_space=pltpu.SMEM),
          pl.BlockSpec(memory_space=pltpu.VMEM),
          pl.BlockSpec(memory_space=pltpu.VMEM),
          pl.BlockSpec(memory_space=pltpu.VMEM),
          pl.BlockSpec(memory_space=pltpu.VMEM),
          pl.BlockSpec(memory_space=pltpu.VMEM),
          pl.BlockSpec(memory_space=pltpu.VMEM),
          pl.BlockSpec(memory_space=pltpu.VMEM),
          pl.BlockSpec(memory_space=pl.ANY),
      ],
      out_specs=pl.BlockSpec(memory_space=pltpu.VMEM),
      scratch_shapes=[
          pltpu.VMEM((2, BLK, 2 * D), jnp.float32),
          pltpu.VMEM((N_PAD, D), jnp.float32),
          pltpu.VMEM((N_PAD, D), jnp.float32),
          pltpu.SemaphoreType.DMA((2,)),
      ],
  )(maxdeg, deg2d, hin, wcat, bias, ws, wn, b, msgs)


def _preprocess(edge_index):
  src = edge_index[0]
  dst = edge_index[1]
  order = jnp.argsort(dst)
  src_s = src[order]
  dst_s = dst[order]
  deg = jnp.bincount(dst_s, length=N_NODES).astype(jnp.int32)
  offsets = jnp.concatenate(
      [jnp.zeros((1,), jnp.int32), jnp.cumsum(deg)[:-1].astype(jnp.int32)])
  perm = jnp.argsort(-deg)
  rank_of = jnp.zeros((N_NODES,), jnp.int32).at[perm].set(
      jnp.arange(N_NODES, dtype=jnp.int32))
  deg_sorted = deg[perm]

  cnt = jnp.bincount(deg, length=N_EDGES + 1)
  n_arr = (N_NODES - jnp.cumsum(cnt)).astype(jnp.int32)
  wsum = jnp.cumsum(jnp.arange(N_EDGES + 1) * cnt).astype(jnp.int32)

  tab_node = (offsets.astype(jnp.uint32) * 16384
              + rank_of.astype(jnp.uint32))
  starts = jnp.where(deg > 0, offsets, N_EDGES)
  seg = jnp.zeros((N_EDGES,), jnp.uint32).at[starts].set(tab_node)
  by_dst = lax.cummax(seg)
  rank_dst = (by_dst % 16384).astype(jnp.int32)
  off_dst = (by_dst // 16384).astype(jnp.int32)
  rank_src = _sc_gather1(rank_of, src_s)
  t_e = jnp.arange(N_EDGES, dtype=jnp.int32) - off_dst
  tab_t = (n_arr.astype(jnp.uint32) * 262144
           + wsum.astype(jnp.uint32))
  by_t = _sc_gather1(tab_t, t_e)
  S_e = t_e * (by_t // 262144).astype(jnp.int32) + (
      by_t % 262144).astype(jnp.int32)
  p_e = S_e + rank_dst

  p_pad = jnp.concatenate(
      [p_e, jnp.arange(N_EDGES, E_PAD, dtype=jnp.int32)])
  v_pad = jnp.concatenate(
      [rank_src, jnp.zeros((E_PAD - N_EDGES,), jnp.int32)])
  g = _make_sc_scatter(E_PAD, 1280, E_PAD)(v_pad, p_pad)
  maxdeg = jnp.max(deg).reshape(1)
  deg2d = jnp.zeros((N_PAD,), jnp.int32).at[:N_NODES].set(
      deg_sorted).reshape(N_PAD // 128, 128)
  return g, perm, rank_of, deg2d, maxdeg


def _pad_params(Wi, Wh, bi, bh, Ws, Wn, b):
  din = Wi.shape[1]
  dout = Ws.shape[0]
  wcat = jnp.zeros((2 * D, G), jnp.float32)
  bias = jnp.zeros((G,), jnp.float32)
  for gi in range(4):
    wcat = wcat.at[0:din, gi * D:gi * D + din].set(
        Wi[gi * din:(gi + 1) * din, :].T)
    wcat = wcat.at[D:D + din, gi * D:gi * D + din].set(
        Wh[gi * din:(gi + 1) * din, :].T)
    bias = bias.at[gi * D:gi * D + din].set(
        bi[gi * din:(gi + 1) * din] + bh[gi * din:(gi + 1) * din])
  ws = jnp.zeros((D, D), jnp.float32).at[:din, :dout].set(Ws.T)
  wn = jnp.zeros((D, D), jnp.float32).at[:din, :dout].set(Wn.T)
  bp = jnp.zeros((D,), jnp.float32).at[:dout].set(b)
  return (wcat.astype(jnp.bfloat16), bias.reshape(1, G),
          ws, wn, bp.reshape(1, D))


_NW = 32


@functools.cache
def _make_sc_gather(B, CH, Dr, dtype, translate=False):
  b_per_w = B // _NW
  nch = b_per_w // CH
  assert B % _NW == 0 and b_per_w % CH == 0 and CH % 16 == 0
  mesh = plsc.VectorSubcoreMesh(core_axis_name="c", subcore_axis_name="s")

  scratch = [
      pltpu.VMEM((CH,), jnp.int32),
      pltpu.VMEM((CH,), jnp.int32),
      pltpu.VMEM((CH, Dr), dtype),
      pltpu.VMEM((CH, Dr), dtype),
      pltpu.SemaphoreType.DMA,
      pltpu.SemaphoreType.DMA,
  ]
  if translate:
    scratch.extend([pltpu.VMEM((N_PAD,), jnp.int32),
                    pltpu.VMEM((CH,), jnp.int32),
                    pltpu.VMEM((CH,), jnp.int32),
                    pltpu.SemaphoreType.DMA])

  @functools.partial(
      pl.kernel, mesh=mesh,
      out_type=jax.ShapeDtypeStruct((B, Dr), dtype),
      scratch_types=scratch,
  )
  def k(table_hbm, idx_hbm, *rest):
    if translate:
      (trans_hbm, out_hbm, idx0, idx1, rows0, rows1, sem0, sem1, tab_v,
       tr0, tr1, semt) = rest
      tr_v = [tr0, tr1]
    else:
      out_hbm, idx0, idx1, rows0, rows1, sem0, sem1 = rest
      tab_v = None
    wid = lax.axis_index("s") * 2 + lax.axis_index("c")
    base = wid * b_per_w
    idx_v = [idx0, idx1]
    rows_v = [rows0, rows1]
    sem = [sem0, sem1]
    if translate:
      pltpu.sync_copy(trans_hbm, tab_v)

    def start_gather(c):
      s = c % 2
      pltpu.sync_copy(idx_hbm.at[pl.ds(base + c * CH, CH)], idx_v[s])
      src_idx = idx_v[s]
      if translate:
        cp = pltpu.make_async_copy(tab_v.at[idx_v[s]], tr_v[s], semt)
        cp.start()
        cp.wait()
        src_idx = tr_v[s]
      pltpu.make_async_copy(table_hbm.at[src_idx], rows_v[s],
                            sem[s]).start()

    start_gather(0)
    for c in range(nch):
      s = c % 2
      if c + 1 < nch:
        start_gather(c + 1)
      pltpu.make_async_copy(table_hbm.at[idx_v[s]], rows_v[s],
                            sem[s]).wait()
      pltpu.sync_copy(rows_v[s], out_hbm.at[pl.ds(base + c * CH, CH)])

  return k


@functools.cache
def _make_sc_scatter(B, CH, M):
  b_per_w = B // _NW
  nch = b_per_w // CH
  assert B % _NW == 0 and b_per_w % CH == 0 and CH % 8 == 0
  mesh = plsc.VectorSubcoreMesh(core_axis_name="c", subcore_axis_name="s")

  @functools.partial(
      pl.kernel, mesh=mesh,
      out_type=jax.ShapeDtypeStruct((M,), jnp.int32),
      scratch_types=[
          pltpu.VMEM((CH,), jnp.int32),
          pltpu.VMEM((CH,), jnp.int32),
          pltpu.VMEM((CH,), jnp.int32),
          pltpu.VMEM((CH,), jnp.int32),
          pltpu.SemaphoreType.DMA,
          pltpu.SemaphoreType.DMA,
      ],
  )
  def k(vals_hbm, pos_hbm, out_hbm, pos0, pos1, rows0, rows1, sem0, sem1):
    wid = lax.axis_index("s") * 2 + lax.axis_index("c")
    base = wid * b_per_w
    pos_v = [pos0, pos1]
    rows_v = [rows0, rows1]
    sem = [sem0, sem1]

    def start_scatter(c):
      s = c % 2
      pltpu.sync_copy(pos_hbm.at[pl.ds(base + c * CH, CH)], pos_v[s])
      pltpu.sync_copy(vals_hbm.at[pl.ds(base + c * CH, CH)], rows_v[s])
      pltpu.make_async_copy(rows_v[s], out_hbm.at[pos_v[s]], sem[s]).start()

    start_scatter(0)
    for c in range(nch):
      s = c % 2
      if c + 1 < nch:
        start_scatter(c + 1)
      pltpu.make_async_copy(rows_v[s], out_hbm.at[pos_v[s]], sem[s]).wait()

  return k


def _gather_rows(table, idx, trans=None):
  if trans is None:
    return _make_sc_gather(idx.shape[0], 320, D, jnp.float32)(table, idx)
  return _make_sc_gather(idx.shape[0], 320, D, jnp.float32, True)(
      table, idx, trans)


@functools.cache
def _make_sc_gather1(B, CH, dtype):
  b_per_w = B // _NW
  nch = b_per_w // CH
  assert B % _NW == 0 and b_per_w % CH == 0 and CH % 8 == 0
  mesh = plsc.VectorSubcoreMesh(core_axis_name="c", subcore_axis_name="s")

  @functools.partial(
      pl.kernel, mesh=mesh,
      out_type=jax.ShapeDtypeStruct((B,), dtype),
      scratch_types=[
          pltpu.VMEM((CH,), jnp.int32),
          pltpu.VMEM((CH,), jnp.int32),
          pltpu.VMEM((CH,), dtype),
          pltpu.VMEM((CH,), dtype),
          pltpu.SemaphoreType.DMA,
          pltpu.SemaphoreType.DMA,
      ],
  )
  def k(table_hbm, idx_hbm, out_hbm, idx0, idx1, rows0, rows1, sem0, sem1):
    wid = lax.axis_index("s") * 2 + lax.axis_index("c")
    base = wid * b_per_w
    idx_v = [idx0, idx1]
    rows_v = [rows0, rows1]
    sem = [sem0, sem1]

    def start_gather(c):
      s = c % 2
      pltpu.sync_copy(idx_hbm.at[pl.ds(base + c * CH, CH)], idx_v[s])
      pltpu.make_async_copy(table_hbm.at[idx_v[s]], rows_v[s],
                            sem[s]).start()

    start_gather(0)
    for c in range(nch):
      s = c % 2
      if c + 1 < nch:
        start_gather(c + 1)
      pltpu.make_async_copy(table_hbm.at[idx_v[s]], rows_v[s],
                            sem[s]).wait()
      pltpu.sync_copy(rows_v[s], out_hbm.at[pl.ds(base + c * CH, CH)])

  return k


def _sc_gather1(table, idx, ch=1000):
  return _make_sc_gather1(idx.shape[0], ch, table.dtype)(table, idx)


def kernel(x, edge_index, l1_Wi, l1_Wh, l1_bi, l1_bh, l1_Ws, l1_Wn, l1_b,
           l2_Wi, l2_Wh, l2_bi, l2_bh, l2_Ws, l2_Wn, l2_b,
           l3_Wi, l3_Wh, l3_bi, l3_bh, l3_Ws, l3_Wn, l3_b,
           l4_Wi, l4_Wh, l4_bi, l4_bh, l4_Ws, l4_Wn, l4_b):
  g, perm, rank_of, deg2d, maxdeg = _preprocess(edge_index)

  params = [
      _pad_params(l1_Wi, l1_Wh, l1_bi, l1_bh, l1_Ws, l1_Wn, l1_b),
      _pad_params(l2_Wi, l2_Wh, l2_bi, l2_bh, l2_Ws, l2_Wn, l2_b),
      _pad_params(l3_Wi, l3_Wh, l3_bi, l3_bh, l3_Ws, l3_Wn, l3_b),
      _pad_params(l4_Wi, l4_Wh, l4_bi, l4_bh, l4_Ws, l4_Wn, l4_b),
  ]
  acts = ["relu", "relu", "relu", "sigmoid"]

  perm_pad = jnp.zeros((N_PAD,), jnp.int32).at[:N_NODES].set(perm)
  rank_pad = jnp.zeros((N_PAD,), jnp.int32).at[:N_NODES].set(rank_of)
  h = jnp.zeros((N_PAD, D), jnp.float32).at[:N_NODES, :x.shape[1]].set(x)
  h = _gather_rows(h, perm_pad)

  for li in range(4):
    wcat, bias, ws, wn, bp = params[li]
    msgs = _gather_rows(h, g)
    h = _run_layer(msgs, h, wcat, bias, ws, wn, bp, deg2d, maxdeg, acts[li])

  out = _gather_rows(h, rank_pad)
  return out[:N_NODES, :1]
